# baseline (speedup 1.0000x reference)
import functools

import jax
import jax.numpy as jnp
from jax import lax
from jax.experimental import pallas as pl
from jax.experimental.pallas import tpu as pltpu
from jax.experimental.pallas import tpu_sc as plsc

N = 10000
E = 320000
F0 = 128
F1 = 128
F2 = 64
F3 = 16

NC = 2
NS = 16
NW = NC * NS
L = 16

CH = 128
NCHK = 80
EPT = NCHK * CH
EPAD = NW * EPT
NP = 10240
RPT = NP // NS

_mesh = plsc.VectorSubcoreMesh(core_axis_name="c", subcore_axis_name="s")


def _deg_body(dst_hbm, ew_hbm, out_hbm, dst_v, ew_v, vbuf, acc, sem):
    del sem
    cid = lax.axis_index("c")
    sid = lax.axis_index("s")
    wid = cid * NS + sid
    pltpu.sync_copy(dst_hbm.at[wid], dst_v)
    pltpu.sync_copy(ew_hbm.at[wid], ew_v)

    zeros16 = jnp.zeros((L,), jnp.float32)

    def zb(r, c):
        for f in range(F1 // L):
            vbuf[r, pl.ds(f * L, L)] = zeros16
        return c

    lax.fori_loop(0, CH, zb, 0)

    row0 = sid * RPT
    for b in range(RPT // CH):
        pltpu.sync_copy(vbuf, acc.at[pl.ds(row0 + b * CH, CH)])
    plsc.subcore_barrier()

    def chunk(j, c):
        def ebody(i, c2):
            e0 = i * L
            ewv = ew_v[j, pl.ds(e0, L)]
            for k in range(L):
                vbuf[e0 + k, pl.ds(0, L)] = jnp.broadcast_to(ewv[k], (L,))
            return c2

        lax.fori_loop(0, CH // L, ebody, 0)
        pltpu.sync_copy(vbuf, acc.at[dst_v.at[j]], add=True)
        return c

    lax.fori_loop(0, NCHK, chunk, 0)
    plsc.subcore_barrier()

    for b in range(RPT // CH):
        r = row0 + b * CH
        pltpu.sync_copy(acc.at[pl.ds(r, CH)], out_hbm.at[cid, pl.ds(r, CH)])


_deg_call = functools.partial(
    pl.kernel,
    out_type=jax.ShapeDtypeStruct((NC, NP, F1), jnp.float32),
    mesh=_mesh,
    scratch_types=[
        pltpu.VMEM((NCHK, CH), jnp.int32),
        pltpu.VMEM((NCHK, CH), jnp.float32),
        pltpu.VMEM((CH, F1), jnp.float32),
        pltpu.VMEM_SHARED((NP, F1), jnp.float32),
        pltpu.SemaphoreType.DMA,
    ],
)(_deg_body)


def _conv_body(F, h_hbm, src_hbm, dst_hbm, ew_hbm, out_hbm,
               src_v, dst_v, ew_v, gbuf, acc, sem):
    cid = lax.axis_index("c")
    sid = lax.axis_index("s")
    wid = cid * NS + sid
    pltpu.sync_copy(src_hbm.at[wid], src_v)
    pltpu.sync_copy(dst_hbm.at[wid], dst_v)
    pltpu.sync_copy(ew_hbm.at[wid], ew_v)

    nf = F // L
    zeros16 = jnp.zeros((L,), jnp.float32)

    def zb(r, c):
        for f in range(nf):
            gbuf[r, pl.ds(f * L, L)] = zeros16
        return c

    lax.fori_loop(0, CH, zb, 0)

    row0 = sid * RPT
    for b in range(RPT // CH):
        pltpu.sync_copy(gbuf, acc.at[pl.ds(row0 + b * CH, CH)])
    plsc.subcore_barrier()

    def chunk(j, c):
        pltpu.async_copy(h_hbm.at[src_v.at[j]], gbuf, sem).wait()

        def edge16(i, c2):
            e0 = i * L
            ewv = ew_v[j, pl.ds(e0, L)]
            for k in range(L):
                e = e0 + k
                wv = jnp.broadcast_to(ewv[k], (L,))
                for f in range(nf):
                    sl = pl.ds(f * L, L)
                    gbuf[e, sl] = gbuf[e, sl] * wv
            return c2

        lax.fori_loop(0, CH // L, edge16, 0)
        pltpu.sync_copy(gbuf, acc.at[dst_v.at[j]], add=True)
        return c

    lax.fori_loop(0, NCHK, chunk, 0)
    plsc.subcore_barrier()

    for b in range(RPT // CH):
        r = row0 + b * CH
        pltpu.sync_copy(acc.at[pl.ds(r, CH)], out_hbm.at[cid, pl.ds(r, CH)])


def _make_conv_call(F):
    return functools.partial(
        pl.kernel,
        out_type=jax.ShapeDtypeStruct((NC, NP, F), jnp.float32),
        mesh=_mesh,
        scratch_types=[
            pltpu.VMEM((NCHK, CH), jnp.int32),
            pltpu.VMEM((NCHK, CH), jnp.int32),
            pltpu.VMEM((NCHK, CH), jnp.float32),
            pltpu.VMEM((CH, F), jnp.float32),
            pltpu.VMEM_SHARED((NP, F), jnp.float32),
            pltpu.SemaphoreType.DMA,
        ],
    )(functools.partial(_conv_body, F))


_conv_call_f1 = _make_conv_call(F1)

BM = 1024


def _dis_from_partials(dp):
    deg = 1.0 + dp[0, :, 0] + dp[1, :, 0]
    return lax.rsqrt(deg)


def _mm1_body(x_ref, w_ref, dp_ref, out_ref):
    dis = _dis_from_partials(dp_ref[...])
    h = jnp.dot(x_ref[...], w_ref[...], preferred_element_type=jnp.float32)
    out_ref[...] = h * dis[:, None]


def _mid_body(s_ref, hp_ref, dp_ref, w2_ref, b1_ref, out_ref):
    dis = _dis_from_partials(dp_ref[...])
    s = s_ref[0] + s_ref[1] + hp_ref[...]
    u = jnp.maximum(dis[:, None] * s + b1_ref[0][None, :], 0.0)
    h2 = jnp.dot(u, w2_ref[...], preferred_element_type=jnp.float32)
    out_ref[...] = h2 * dis[:, None]


def _final_body(s_ref, hp_ref, dp_ref, w3_ref, b2_ref, b3_ref, out_ref):
    dis = _dis_from_partials(dp_ref[...])
    agg = (s_ref[0] + s_ref[1] + hp_ref[...])[:, :F2]
    v = dis[:, None] * agg + b2_ref[0][None, :]
    z = jnp.dot(v, w3_ref[...], preferred_element_type=jnp.float32)
    out_ref[...] = jax.nn.sigmoid(z + b3_ref[0][None, :])


def _row_spec(F):
    return pl.BlockSpec((BM, F), lambda i: (i, 0))


def _part_spec(F):
    return pl.BlockSpec((NC, BM, F), lambda i: (0, i, 0))


def _full_spec(a, b):
    return pl.BlockSpec((a, b), lambda i: (0, 0))


def kernel(x, edge_index, edge_attr, W1, b1, W2, b2, W3, b3):
    src = edge_index[0]
    dst = edge_index[1]
    pad = EPAD - E
    pad_dst = N + (jnp.arange(pad, dtype=dst.dtype) % (NP - N))
    src3 = jnp.pad(src, (0, pad)).reshape(NW, NCHK, CH)
    dst3 = jnp.concatenate([dst, pad_dst]).reshape(NW, NCHK, CH)
    ew3 = jnp.pad(edge_attr, (0, pad)).reshape(NW, NCHK, CH)
    x_p = jnp.pad(x, ((0, NP - N), (0, 0)))

    degp = _deg_call(dst3, ew3)

    h1p = pl.pallas_call(
        _mm1_body,
        grid=(NP // BM,),
        in_specs=[_row_spec(F0), _full_spec(F0, F1), _part_spec(F1)],
        out_specs=_row_spec(F1),
        out_shape=jax.ShapeDtypeStruct((NP, F1), jnp.float32),
    )(x_p, W1, degp)

    s1 = _conv_call_f1(h1p, src3, dst3, ew3)

    W2p = jnp.pad(W2, ((0, 0), (0, F1 - F2)))
    h2p = pl.pallas_call(
        _mid_body,
        grid=(NP // BM,),
        in_specs=[_part_spec(F1), _row_spec(F1), _part_spec(F1),
                  _full_spec(F1, F1), _full_spec(1, F1)],
        out_specs=_row_spec(F1),
        out_shape=jax.ShapeDtypeStruct((NP, F1), jnp.float32),
    )(s1, h1p, degp, W2p, b1.reshape(1, F1))

    s2 = _conv_call_f1(h2p, src3, dst3, ew3)

    out = pl.pallas_call(
        _final_body,
        grid=(NP // BM,),
        in_specs=[_part_spec(F1), _row_spec(F1), _part_spec(F1),
                  _full_spec(F2, F3), _full_spec(1, F2), _full_spec(1, F3)],
        out_specs=_row_spec(F3),
        out_shape=jax.ShapeDtypeStruct((NP, F3), jnp.float32),
    )(s2, h2p, degp, W3, b2.reshape(1, F2), b3.reshape(1, F3))

    return out[:N]


# speedup vs baseline: 2.4008x; 2.4008x over previous
import functools

import jax
import jax.numpy as jnp
from jax import lax
from jax.experimental import pallas as pl
from jax.experimental.pallas import tpu as pltpu
from jax.experimental.pallas import tpu_sc as plsc

N = 10000
E = 320000
F0 = 128
F1 = 128
F2 = 64
F3 = 16

NC = 2
NS = 16
NW = NC * NS
L = 16

CH = 128
NCHK = 80
EPT = NCHK * CH
EPAD = NW * EPT
NP = 10240
RPT = NP // NS

_mesh = plsc.VectorSubcoreMesh(core_axis_name="c", subcore_axis_name="s")


def _deg_body(dst_hbm, ew_hbm, out_hbm, dst_v, ew_v, vbuf, acc, sem):
    del sem
    cid = lax.axis_index("c")
    sid = lax.axis_index("s")
    wid = cid * NS + sid
    pltpu.sync_copy(dst_hbm.at[wid], dst_v)
    pltpu.sync_copy(ew_hbm.at[wid], ew_v)

    zeros16 = jnp.zeros((L,), jnp.float32)

    def zb(r, c):
        for f in range(F1 // L):
            vbuf[r, pl.ds(f * L, L)] = zeros16
        return c

    lax.fori_loop(0, CH, zb, 0)

    row0 = sid * RPT
    for b in range(RPT // CH):
        pltpu.sync_copy(vbuf, acc.at[pl.ds(row0 + b * CH, CH)])
    plsc.subcore_barrier()

    def chunk(j, c):
        def ebody(i, c2):
            e0 = i * L
            ewv = ew_v[j, pl.ds(e0, L)]
            for k in range(L):
                vbuf[e0 + k, pl.ds(0, L)] = jnp.broadcast_to(ewv[k], (L,))
            return c2

        lax.fori_loop(0, CH // L, ebody, 0)
        pltpu.sync_copy(vbuf, acc.at[dst_v.at[j]], add=True)
        return c

    lax.fori_loop(0, NCHK, chunk, 0)
    plsc.subcore_barrier()

    for b in range(RPT // CH):
        r = row0 + b * CH
        pltpu.sync_copy(acc.at[pl.ds(r, CH)], out_hbm.at[cid, pl.ds(r, CH)])


_deg_call = functools.partial(
    pl.kernel,
    out_type=jax.ShapeDtypeStruct((NC, NP, F1), jnp.float32),
    mesh=_mesh,
    scratch_types=[
        pltpu.VMEM((NCHK, CH), jnp.int32),
        pltpu.VMEM((NCHK, CH), jnp.float32),
        pltpu.VMEM((CH, F1), jnp.float32),
        pltpu.VMEM_SHARED((NP, F1), jnp.float32),
        pltpu.SemaphoreType.DMA,
    ],
)(_deg_body)


def _conv_body(F, h_hbm, src_hbm, dst_hbm, ew_hbm, out_hbm,
               src_v, dst_v, ew_v, gbuf, acc, sem):
    cid = lax.axis_index("c")
    sid = lax.axis_index("s")
    wid = cid * NS + sid
    pltpu.sync_copy(src_hbm.at[wid], src_v)
    pltpu.sync_copy(dst_hbm.at[wid], dst_v)
    pltpu.sync_copy(ew_hbm.at[wid], ew_v)

    nf = F // L
    zeros16 = jnp.zeros((L,), jnp.float32)

    def zb(r, c):
        for f in range(nf):
            gbuf[r, pl.ds(f * L, L)] = zeros16
        return c

    lax.fori_loop(0, CH, zb, 0)

    row0 = sid * RPT
    for b in range(RPT // CH):
        pltpu.sync_copy(gbuf, acc.at[pl.ds(row0 + b * CH, CH)])
    plsc.subcore_barrier()

    def chunk(j, c):
        pltpu.async_copy(h_hbm.at[src_v.at[j]], gbuf, sem).wait()

        def edge16(i, c2):
            e0 = i * L
            ewv = ew_v[j, pl.ds(e0, L)]
            for k in range(L):
                e = e0 + k
                wv = jnp.broadcast_to(ewv[k], (L,))
                for f in range(nf):
                    sl = pl.ds(f * L, L)
                    gbuf[e, sl] = gbuf[e, sl] * wv
            return c2

        lax.fori_loop(0, CH // L, edge16, 0)
        pltpu.sync_copy(gbuf, acc.at[dst_v.at[j]], add=True)
        return c

    lax.fori_loop(0, NCHK, chunk, 0)
    plsc.subcore_barrier()

    for b in range(RPT // CH):
        r = row0 + b * CH
        pltpu.sync_copy(acc.at[pl.ds(r, CH)], out_hbm.at[cid, pl.ds(r, CH)])


def _make_conv_call(F):
    return functools.partial(
        pl.kernel,
        out_type=jax.ShapeDtypeStruct((NC, NP, F), jnp.float32),
        mesh=_mesh,
        scratch_types=[
            pltpu.VMEM((NCHK, CH), jnp.int32),
            pltpu.VMEM((NCHK, CH), jnp.int32),
            pltpu.VMEM((NCHK, CH), jnp.float32),
            pltpu.VMEM((CH, F), jnp.float32),
            pltpu.VMEM_SHARED((NP, F), jnp.float32),
            pltpu.SemaphoreType.DMA,
        ],
    )(functools.partial(_conv_body, F))


_conv_call_f1 = _make_conv_call(F1)

BM = 1024


def _dis_from_partials(dp):
    deg = 1.0 + dp[0, :, 0] + dp[1, :, 0]
    return lax.rsqrt(deg)


def _mm1_body(x_ref, w_ref, dp_ref, out_ref):
    dis = _dis_from_partials(dp_ref[...])
    h = jnp.dot(x_ref[...], w_ref[...], preferred_element_type=jnp.float32)
    out_ref[...] = h * dis[:, None]


def _mid_body(s_ref, hp_ref, dp_ref, w2_ref, b1_ref, out_ref):
    dis = _dis_from_partials(dp_ref[...])
    s = s_ref[0] + s_ref[1] + hp_ref[...]
    u = jnp.maximum(dis[:, None] * s + b1_ref[0][None, :], 0.0)
    h2 = jnp.dot(u, w2_ref[...], preferred_element_type=jnp.float32)
    out_ref[...] = h2 * dis[:, None]


def _final_body(s_ref, hp_ref, dp_ref, w3_ref, b2_ref, b3_ref, out_ref):
    dis = _dis_from_partials(dp_ref[...])
    agg = (s_ref[0] + s_ref[1] + hp_ref[...])[:, :F2]
    v = dis[:, None] * agg + b2_ref[0][None, :]
    z = jnp.dot(v, w3_ref[...], preferred_element_type=jnp.float32)
    out_ref[...] = jax.nn.sigmoid(z + b3_ref[0][None, :])


def _row_spec(F):
    return pl.BlockSpec((BM, F), lambda i: (i, 0))


def _part_spec(F):
    return pl.BlockSpec((NC, BM, F), lambda i: (0, i, 0))


def _full_spec(a, b):
    return pl.BlockSpec((a, b), lambda i: (0, 0))


def kernel(x, edge_index, edge_attr, W1, b1, W2, b2, W3, b3):
    src = edge_index[0]
    dst = edge_index[1]
    pad = EPAD - E
    pad_dst = N + (jnp.arange(pad, dtype=dst.dtype) % (NP - N))
    pad_src = jnp.arange(pad, dtype=src.dtype) % NP
    src3 = jnp.concatenate([src, pad_src]).reshape(NW, NCHK, CH)
    dst3 = jnp.concatenate([dst, pad_dst]).reshape(NW, NCHK, CH)
    ew3 = jnp.pad(edge_attr, (0, pad)).reshape(NW, NCHK, CH)
    x_p = jnp.pad(x, ((0, NP - N), (0, 0)))

    degp = _deg_call(dst3, ew3)

    h1p = pl.pallas_call(
        _mm1_body,
        grid=(NP // BM,),
        in_specs=[_row_spec(F0), _full_spec(F0, F1), _part_spec(F1)],
        out_specs=_row_spec(F1),
        out_shape=jax.ShapeDtypeStruct((NP, F1), jnp.float32),
    )(x_p, W1, degp)

    s1 = _conv_call_f1(h1p, src3, dst3, ew3)

    W2p = jnp.pad(W2, ((0, 0), (0, F1 - F2)))
    h2p = pl.pallas_call(
        _mid_body,
        grid=(NP // BM,),
        in_specs=[_part_spec(F1), _row_spec(F1), _part_spec(F1),
                  _full_spec(F1, F1), _full_spec(1, F1)],
        out_specs=_row_spec(F1),
        out_shape=jax.ShapeDtypeStruct((NP, F1), jnp.float32),
    )(s1, h1p, degp, W2p, b1.reshape(1, F1))

    s2 = _conv_call_f1(h2p, src3, dst3, ew3)

    out = pl.pallas_call(
        _final_body,
        grid=(NP // BM,),
        in_specs=[_part_spec(F1), _row_spec(F1), _part_spec(F1),
                  _full_spec(F2, F3), _full_spec(1, F2), _full_spec(1, F3)],
        out_specs=_row_spec(F3),
        out_shape=jax.ShapeDtypeStruct((NP, F3), jnp.float32),
    )(s2, h2p, degp, W3, b2.reshape(1, F2), b3.reshape(1, F3))

    return out[:N]


# speedup vs baseline: 3.2559x; 1.3561x over previous
import functools

import jax
import jax.numpy as jnp
from jax import lax
from jax.experimental import pallas as pl
from jax.experimental.pallas import tpu as pltpu
from jax.experimental.pallas import tpu_sc as plsc

N = 10000
E = 320000
F0 = 128
F1 = 128
F2 = 64
F3 = 16

NC = 2
NS = 16
NW = NC * NS
L = 16

CH = 128
NCHK = 80
QC = 16
EPT = NCHK * CH
EPAD = NW * EPT
NP = 10240
RPT = NP // NS

_mesh = plsc.VectorSubcoreMesh(core_axis_name="c", subcore_axis_name="s")


def _deg_body(dst_hbm, ew_hbm, out_hbm, dst_v, ew_v, vbuf, acc, sem):
    del sem
    cid = lax.axis_index("c")
    sid = lax.axis_index("s")
    wid = cid * NS + sid
    pltpu.sync_copy(dst_hbm.at[wid], dst_v)
    pltpu.sync_copy(ew_hbm.at[wid], ew_v)

    zeros16 = jnp.zeros((L,), jnp.float32)

    def zb(r, c):
        for f in range(F1 // L):
            vbuf[r, pl.ds(f * L, L)] = zeros16
        return c

    lax.fori_loop(0, CH, zb, 0)

    row0 = sid * RPT
    for b in range(RPT // CH):
        pltpu.sync_copy(vbuf, acc.at[pl.ds(row0 + b * CH, CH)])
    plsc.subcore_barrier()

    def chunk(j, c):
        def ebody(i, c2):
            e0 = i * L
            ewv = ew_v[j, pl.ds(e0, L)]
            for k in range(L):
                vbuf[e0 + k, pl.ds(0, L)] = jnp.broadcast_to(ewv[k], (L,))
            return c2

        lax.fori_loop(0, CH // L, ebody, 0)
        pltpu.sync_copy(vbuf, acc.at[dst_v.at[j]], add=True)
        return c

    lax.fori_loop(0, NCHK, chunk, 0)
    plsc.subcore_barrier()

    for b in range(RPT // CH):
        r = row0 + b * CH
        pltpu.sync_copy(acc.at[pl.ds(r, CH)], out_hbm.at[cid, pl.ds(r, CH)])


_deg_call = functools.partial(
    pl.kernel,
    out_type=jax.ShapeDtypeStruct((NC, NP, F1), jnp.float32),
    mesh=_mesh,
    scratch_types=[
        pltpu.VMEM((NCHK, CH), jnp.int32),
        pltpu.VMEM((NCHK, CH), jnp.float32),
        pltpu.VMEM((CH, F1), jnp.float32),
        pltpu.VMEM_SHARED((NP, F1), jnp.float32),
        pltpu.SemaphoreType.DMA,
    ],
)(_deg_body)


def _conv_body(F, nf_valid, h_hbm, src_hbm, dst_hbm, ew_hbm, out_hbm,
               src_v, dst_v, ew_v, gbufA, gbufB, acc, semA, semB):
    cid = lax.axis_index("c")
    sid = lax.axis_index("s")
    wid = cid * NS + sid

    nf = F // L
    zeros16 = jnp.zeros((L,), jnp.float32)

    def zb(r, c):
        for f in range(nf):
            gbufA[r, pl.ds(f * L, L)] = zeros16
        return c

    lax.fori_loop(0, CH, zb, 0)

    row0 = sid * RPT
    for b in range(RPT // CH):
        pltpu.sync_copy(gbufA, acc.at[pl.ds(row0 + b * CH, CH)])
    plsc.subcore_barrier()

    bufs = (gbufA, gbufB)
    sems = (semA, semB)

    def do_chunk(buf, j):
        def edge16(i, c2):
            e0 = i * L
            ewv = ew_v[j, pl.ds(e0, L)]
            for k in range(L):
                e = e0 + k
                wv = jnp.broadcast_to(ewv[k], (L,))
                for f in range(nf_valid):
                    sl = pl.ds(f * L, L)
                    buf[e, sl] = buf[e, sl] * wv
            return c2

        lax.fori_loop(0, CH // L, edge16, 0)
        pltpu.sync_copy(buf, acc.at[dst_v.at[j]], add=True)

    for q in range(NCHK // QC):
        q0 = q * QC
        pltpu.sync_copy(src_hbm.at[wid, pl.ds(q0, QC)], src_v)
        pltpu.sync_copy(dst_hbm.at[wid, pl.ds(q0, QC)], dst_v)
        pltpu.sync_copy(ew_hbm.at[wid, pl.ds(q0, QC)], ew_v)

        pltpu.async_copy(h_hbm.at[src_v.at[0]], gbufA, semA)
        pltpu.async_copy(h_hbm.at[src_v.at[1]], gbufB, semB)

        def pair(p, c):
            for b in range(2):
                j = 2 * p + b
                pltpu.make_async_copy(
                    h_hbm.at[src_v.at[j]], bufs[b], sems[b]).wait()
                do_chunk(bufs[b], j)
                jn = jnp.minimum(j + 2, QC - 1)
                pltpu.async_copy(h_hbm.at[src_v.at[jn]], bufs[b], sems[b])
            return c

        lax.fori_loop(0, QC // 2, pair, 0)
        pltpu.make_async_copy(h_hbm.at[src_v.at[QC - 1]], gbufA, semA).wait()
        pltpu.make_async_copy(h_hbm.at[src_v.at[QC - 1]], gbufB, semB).wait()
    plsc.subcore_barrier()

    for b in range(RPT // CH):
        r = row0 + b * CH
        pltpu.sync_copy(acc.at[pl.ds(r, CH)], out_hbm.at[cid, pl.ds(r, CH)])


def _make_conv_call(F, nf_valid):
    return functools.partial(
        pl.kernel,
        out_type=jax.ShapeDtypeStruct((NC, NP, F), jnp.float32),
        mesh=_mesh,
        scratch_types=[
            pltpu.VMEM((QC, CH), jnp.int32),
            pltpu.VMEM((QC, CH), jnp.int32),
            pltpu.VMEM((QC, CH), jnp.float32),
            pltpu.VMEM((CH, F), jnp.float32),
            pltpu.VMEM((CH, F), jnp.float32),
            pltpu.VMEM_SHARED((NP, F), jnp.float32),
            pltpu.SemaphoreType.DMA,
            pltpu.SemaphoreType.DMA,
        ],
    )(functools.partial(_conv_body, F, nf_valid))


_conv_call_c1 = _make_conv_call(F1, F1 // L)
_conv_call_c2 = _make_conv_call(F1, F2 // L)

BM = 1024


def _dis_from_partials(dp):
    deg = 1.0 + dp[0, :, 0] + dp[1, :, 0]
    return lax.rsqrt(deg)


def _mm1_body(x_ref, w_ref, dp_ref, out_ref):
    dis = _dis_from_partials(dp_ref[...])
    h = jnp.dot(x_ref[...], w_ref[...], preferred_element_type=jnp.float32)
    out_ref[...] = h * dis[:, None]


def _mid_body(s_ref, hp_ref, dp_ref, w2_ref, b1_ref, out_ref):
    dis = _dis_from_partials(dp_ref[...])
    s = s_ref[0] + s_ref[1] + hp_ref[...]
    u = jnp.maximum(dis[:, None] * s + b1_ref[0][None, :], 0.0)
    h2 = jnp.dot(u, w2_ref[...], preferred_element_type=jnp.float32)
    out_ref[...] = h2 * dis[:, None]


def _final_body(s_ref, hp_ref, dp_ref, w3_ref, b2_ref, b3_ref, out_ref):
    dis = _dis_from_partials(dp_ref[...])
    agg = (s_ref[0] + s_ref[1] + hp_ref[...])[:, :F2]
    v = dis[:, None] * agg + b2_ref[0][None, :]
    z = jnp.dot(v, w3_ref[...], preferred_element_type=jnp.float32)
    out_ref[...] = jax.nn.sigmoid(z + b3_ref[0][None, :])


def _row_spec(F):
    return pl.BlockSpec((BM, F), lambda i: (i, 0))


def _part_spec(F):
    return pl.BlockSpec((NC, BM, F), lambda i: (0, i, 0))


def _full_spec(a, b):
    return pl.BlockSpec((a, b), lambda i: (0, 0))


def kernel(x, edge_index, edge_attr, W1, b1, W2, b2, W3, b3):
    src = edge_index[0]
    dst = edge_index[1]
    pad = EPAD - E
    pad_src = jnp.arange(pad, dtype=src.dtype) % NP
    pad_dst = N + (jnp.arange(pad, dtype=dst.dtype) % (NP - N))
    src3 = jnp.concatenate([src, pad_src]).reshape(NW, NCHK, CH)
    dst3 = jnp.concatenate([dst, pad_dst]).reshape(NW, NCHK, CH)
    ew3 = jnp.pad(edge_attr, (0, pad)).reshape(NW, NCHK, CH)
    x_p = jnp.pad(x, ((0, NP - N), (0, 0)))

    degp = _deg_call(dst3, ew3)

    h1p = pl.pallas_call(
        _mm1_body,
        grid=(NP // BM,),
        in_specs=[_row_spec(F0), _full_spec(F0, F1), _part_spec(F1)],
        out_specs=_row_spec(F1),
        out_shape=jax.ShapeDtypeStruct((NP, F1), jnp.float32),
    )(x_p, W1, degp)

    s1 = _conv_call_c1(h1p, src3, dst3, ew3)

    W2p = jnp.pad(W2, ((0, 0), (0, F1 - F2)))
    h2p = pl.pallas_call(
        _mid_body,
        grid=(NP // BM,),
        in_specs=[_part_spec(F1), _row_spec(F1), _part_spec(F1),
                  _full_spec(F1, F1), _full_spec(1, F1)],
        out_specs=_row_spec(F1),
        out_shape=jax.ShapeDtypeStruct((NP, F1), jnp.float32),
    )(s1, h1p, degp, W2p, b1.reshape(1, F1))

    s2 = _conv_call_c2(h2p, src3, dst3, ew3)

    out = pl.pallas_call(
        _final_body,
        grid=(NP // BM,),
        in_specs=[_part_spec(F1), _row_spec(F1), _part_spec(F1),
                  _full_spec(F2, F3), _full_spec(1, F2), _full_spec(1, F3)],
        out_specs=_row_spec(F3),
        out_shape=jax.ShapeDtypeStruct((NP, F3), jnp.float32),
    )(s2, h2p, degp, W3, b2.reshape(1, F2), b3.reshape(1, F3))

    return out[:N]
